# Initial kernel scaffold; baseline (speedup 1.0000x reference)
import jax
import jax.numpy as jnp
from jax.experimental import pallas as pl


def kernel(x, edge_index, style, trs_w, trs_b, bn2_w, bn2_b, ad1_w1, ad1_b1, ad1_w2, ad1_b2, fc1_w, fc1_b, bn1_w, bn1_b, ad2_w1, ad2_b1, ad2_w2, ad2_b2, gat_w, gat_att_src, gat_att_dst, gat_b, ad3_w1, ad3_b1, ad3_w2, ad3_b2):
    raise NotImplementedError("write your pallas kernel here")


# speedup vs baseline: 59.7798x; 59.7798x over previous
import jax
import jax.numpy as jnp
from jax import lax
from jax.experimental import pallas as pl
from jax.experimental.pallas import tpu as pltpu
from jax.experimental.pallas import tpu_sc as plsc

N_IN, N_OUT, D, H, E = 2048, 8192, 64, 256, 524288
BLK = 512
DP = 128
NC, NS = 2, 16
NW = NC * NS
EPT = E // NW
W = 256
NWIN = EPT // W
RPW = W // 128
RSUB = N_OUT // NS


def _leaky(x):
    return jnp.where(x >= 0, x, 0.2 * x)


def _tc1_body(x_ref, trsw_ref, trsb_ref, bn2w_ref, bn2b_ref, style_ref,
              a1w1_ref, a1b1_ref, a1w2_ref, a1b2_ref, fcw_ref, fcb_ref,
              u_ref, s1_ref, s2_ref):
    y = jnp.dot(trsw_ref[...], x_ref[...], preferred_element_type=jnp.float32)
    y = y + trsb_ref[...][:, None]
    m = jnp.mean(y, axis=1, keepdims=True)
    v = jnp.mean((y - m) ** 2, axis=1, keepdims=True)
    h = (y - m) / jnp.sqrt(v + 1e-5) * bn2w_ref[...][:, None] + bn2b_ref[...][:, None]
    h = _leaky(h)
    rm = jnp.mean(h, axis=1, keepdims=True)
    rs = jnp.sqrt(jnp.sum((h - rm) ** 2, axis=1, keepdims=True) / (D - 1))
    st = style_ref[...]
    gamma = jnp.dot(st, a1w1_ref[...].T, preferred_element_type=jnp.float32) + a1b1_ref[...]
    beta = jnp.dot(st, a1w2_ref[...].T, preferred_element_type=jnp.float32) + a1b2_ref[...]
    h0 = gamma * (h - rm) / (rs + 1e-8) + beta
    u = jnp.dot(h0, fcw_ref[...].T, preferred_element_type=jnp.float32) + fcb_ref[...]
    u_ref[...] = u
    ps1 = jnp.sum(u, axis=0, keepdims=True)
    ps2 = jnp.sum(u * u, axis=0, keepdims=True)

    @pl.when(pl.program_id(0) == 0)
    def _():
        s1_ref[...] = ps1
        s2_ref[...] = ps2

    @pl.when(pl.program_id(0) != 0)
    def _():
        s1_ref[...] = s1_ref[...] + ps1
        s2_ref[...] = s2_ref[...] + ps2


def _tc1(x, trs_w, trs_b, bn2_w, bn2_b, style, ad1_w1, ad1_b1, ad1_w2, ad1_b2,
         fc1_w, fc1_b):
    return pl.pallas_call(
        _tc1_body,
        grid=(N_OUT // BLK,),
        in_specs=[
            pl.BlockSpec((N_IN, D), lambda i: (0, 0)),
            pl.BlockSpec((BLK, N_IN), lambda i: (i, 0)),
            pl.BlockSpec((BLK,), lambda i: (i,)),
            pl.BlockSpec((BLK,), lambda i: (i,)),
            pl.BlockSpec((BLK,), lambda i: (i,)),
            pl.BlockSpec((BLK, D), lambda i: (i, 0)),
            pl.BlockSpec((D, D), lambda i: (0, 0)),
            pl.BlockSpec((D,), lambda i: (0,)),
            pl.BlockSpec((D, D), lambda i: (0, 0)),
            pl.BlockSpec((D,), lambda i: (0,)),
            pl.BlockSpec((H, D), lambda i: (0, 0)),
            pl.BlockSpec((H,), lambda i: (0,)),
        ],
        out_specs=[
            pl.BlockSpec((BLK, H), lambda i: (i, 0)),
            pl.BlockSpec((1, H), lambda i: (0, 0)),
            pl.BlockSpec((1, H), lambda i: (0, 0)),
        ],
        out_shape=[
            jax.ShapeDtypeStruct((N_OUT, H), jnp.float32),
            jax.ShapeDtypeStruct((1, H), jnp.float32),
            jax.ShapeDtypeStruct((1, H), jnp.float32),
        ],
    )(x, trs_w, trs_b, bn2_w, bn2_b, style, ad1_w1, ad1_b1, ad1_w2, ad1_b2,
      fc1_w, fc1_b)


def _tc2_body(u_ref, mu_ref, var_ref, bn1w_ref, bn1b_ref, style_ref,
              a2w1_ref, a2b1_ref, a2w2_ref, a2b2_ref, gatw_ref, asv_ref,
              adv_ref, hp_ref, asrc_ref, adst_ref, amax_ref):
    u = u_ref[...]
    h1 = (u - mu_ref[...]) / jnp.sqrt(var_ref[...] + 1e-5) * bn1w_ref[...] + bn1b_ref[...]
    h1 = _leaky(h1)
    rm = jnp.mean(h1, axis=1, keepdims=True)
    rs = jnp.sqrt(jnp.sum((h1 - rm) ** 2, axis=1, keepdims=True) / (H - 1))
    st = style_ref[...]
    gamma = jnp.dot(st, a2w1_ref[...].T, preferred_element_type=jnp.float32) + a2b1_ref[...]
    beta = jnp.dot(st, a2w2_ref[...].T, preferred_element_type=jnp.float32) + a2b2_ref[...]
    h1n = gamma * (h1 - rm) / (rs + 1e-8) + beta
    hp = jnp.dot(h1n, gatw_ref[...].T, preferred_element_type=jnp.float32)
    one = jnp.ones((hp.shape[0], 1), jnp.float32)
    zer = jnp.zeros((hp.shape[0], DP - D - 1), jnp.float32)
    hp_ref[...] = jnp.concatenate([hp, one, zer], axis=1)
    asrc = jnp.sum(hp * asv_ref[...][None, :], axis=1)
    adst = jnp.sum(hp * adv_ref[...][None, :], axis=1)
    asrc_ref[...] = asrc
    adst_ref[...] = adst
    bm = jnp.max(asrc)

    @pl.when(pl.program_id(0) == 0)
    def _():
        amax_ref[...] = jnp.full((1, 128), bm, jnp.float32)

    @pl.when(pl.program_id(0) != 0)
    def _():
        amax_ref[...] = jnp.maximum(amax_ref[...], bm)


def _tc2(u, mu, var, bn1_w, bn1_b, style, ad2_w1, ad2_b1, ad2_w2, ad2_b2,
         gat_w, att_src, att_dst):
    return pl.pallas_call(
        _tc2_body,
        grid=(N_OUT // BLK,),
        in_specs=[
            pl.BlockSpec((BLK, H), lambda i: (i, 0)),
            pl.BlockSpec((1, H), lambda i: (0, 0)),
            pl.BlockSpec((1, H), lambda i: (0, 0)),
            pl.BlockSpec((H,), lambda i: (0,)),
            pl.BlockSpec((H,), lambda i: (0,)),
            pl.BlockSpec((BLK, D), lambda i: (i, 0)),
            pl.BlockSpec((H, D), lambda i: (0, 0)),
            pl.BlockSpec((H,), lambda i: (0,)),
            pl.BlockSpec((H, D), lambda i: (0, 0)),
            pl.BlockSpec((H,), lambda i: (0,)),
            pl.BlockSpec((D, H), lambda i: (0, 0)),
            pl.BlockSpec((D,), lambda i: (0,)),
            pl.BlockSpec((D,), lambda i: (0,)),
        ],
        out_specs=[
            pl.BlockSpec((BLK, DP), lambda i: (i, 0)),
            pl.BlockSpec((BLK,), lambda i: (i,)),
            pl.BlockSpec((BLK,), lambda i: (i,)),
            pl.BlockSpec((1, 128), lambda i: (0, 0)),
        ],
        out_shape=[
            jax.ShapeDtypeStruct((N_OUT, DP), jnp.float32),
            jax.ShapeDtypeStruct((N_OUT,), jnp.float32),
            jax.ShapeDtypeStruct((N_OUT,), jnp.float32),
            jax.ShapeDtypeStruct((1, 128), jnp.float32),
        ],
    )(u, mu, var, bn1_w, bn1_b, style, ad2_w1, ad2_b1, ad2_w2, ad2_b2,
      gat_w, att_src, att_dst)


def _sc_edge_body(src_hbm, dst_hbm, asrc_hbm, adst_hbm, amax_hbm, hp_hbm,
                  agg_out,
                  src_w, dst_w, asrc_t, adst_t, amax_t, rows_v, agg_sh, sem):
    cid = lax.axis_index("c")
    sid = lax.axis_index("s")
    wid = cid * NS + sid
    z16 = jnp.zeros((16,), jnp.float32)

    pltpu.sync_copy(asrc_hbm, asrc_t)
    pltpu.sync_copy(adst_hbm, adst_t)
    pltpu.sync_copy(amax_hbm, amax_t)

    def _zrow(j, carry):
        for q in range(DP // 16):
            rows_v[j, pl.ds(q * 16, 16)] = z16
        return carry
    lax.fori_loop(0, W, _zrow, 0)

    pltpu.sync_copy(rows_v, agg_sh.at[pl.ds(sid * RSUB, W)])
    pltpu.sync_copy(rows_v, agg_sh.at[pl.ds(sid * RSUB + W, W)])
    plsc.subcore_barrier()
    av = amax_t[...]

    def _window(w, carry):
        rowbase = wid * (EPT // 128) + w * RPW
        pltpu.sync_copy(src_hbm.at[pl.ds(rowbase, RPW)], src_w)
        pltpu.sync_copy(dst_hbm.at[pl.ds(rowbase, RPW)], dst_w)
        cps = [
            pltpu.async_copy(hp_hbm.at[src_w.at[j]],
                             rows_v.at[pl.ds(j * 128, 128)], sem)
            for j in range(RPW)
        ]
        for cp in cps:
            cp.wait()

        for j in range(RPW):
            def _vreg(k2, carry2, j=j):
                o = pl.multiple_of(k2 * 16, 16)
                idx_s = src_w[j, pl.ds(o, 16)]
                idx_d = dst_w[j, pl.ds(o, 16)]
                s = plsc.load_gather(asrc_t, [idx_s])
                t = plsc.load_gather(adst_t, [idx_d])
                stv = s + t
                e = jnp.where(stv >= 0, stv, 0.2 * stv)
                bv = av + t
                b = jnp.where(bv >= 0, bv, 0.2 * bv)
                ex = jnp.exp(e - b)
                base = j * 128 + o
                for l in range(16):
                    wv = jnp.full((16,), ex[l], jnp.float32)
                    for q in range(5):
                        rows_v[base + l, pl.ds(q * 16, 16)] = (
                            rows_v[base + l, pl.ds(q * 16, 16)] * wv)
                return carry2
            lax.fori_loop(0, 8, _vreg, 0)

        for j in range(RPW):
            pltpu.sync_copy(rows_v.at[pl.ds(j * 128, 128)],
                            agg_sh.at[dst_w.at[j]], add=True)
        return carry

    lax.fori_loop(0, NWIN, _window, 0)
    plsc.subcore_barrier()

    out_base = cid * N_OUT + sid * RSUB
    pltpu.sync_copy(agg_sh.at[pl.ds(sid * RSUB, RSUB)],
                    agg_out.at[pl.ds(out_base, RSUB)])


def _sc_edge(src2d, dst2d, a_src, a_dst, amax16, hp_pad):
    mesh = plsc.VectorSubcoreMesh(core_axis_name="c", subcore_axis_name="s",
                                  num_cores=NC, num_subcores=NS)
    return pl.kernel(
        _sc_edge_body,
        out_type=jax.ShapeDtypeStruct((NC * N_OUT, DP), jnp.float32),
        mesh=mesh,
        compiler_params=pltpu.CompilerParams(needs_layout_passes=False),
        scratch_types=[
            pltpu.VMEM((RPW, 128), jnp.int32),
            pltpu.VMEM((RPW, 128), jnp.int32),
            pltpu.VMEM((N_OUT,), jnp.float32),
            pltpu.VMEM((N_OUT,), jnp.float32),
            pltpu.VMEM((16,), jnp.float32),
            pltpu.VMEM((W, DP), jnp.float32),
            pltpu.VMEM_SHARED((N_OUT, DP), jnp.float32),
            pltpu.SemaphoreType.DMA,
        ],
    )(src2d, dst2d, a_src, a_dst, amax16, hp_pad)


def _tc3_body(agg0_ref, agg1_ref, hp_ref, asrc_ref, adst_ref, btab_ref,
              gatb_ref, style_ref, a3w1_ref, a3b1_ref, a3w2_ref, a3b2_ref,
              out_ref):
    es = _leaky(asrc_ref[...] + adst_ref[...])
    ex_self = jnp.exp(es - btab_ref[...])
    hp = hp_ref[...][:, :D]
    den = agg0_ref[...][:, D] + agg1_ref[...][:, D] + ex_self
    aggr = agg0_ref[...][:, :D] + agg1_ref[...][:, :D] + ex_self[:, None] * hp
    agg = aggr / (den[:, None] + 1e-16) + gatb_ref[...]
    h2 = _leaky(agg)
    rm = jnp.mean(h2, axis=1, keepdims=True)
    rs = jnp.sqrt(jnp.sum((h2 - rm) ** 2, axis=1, keepdims=True) / (D - 1))
    st = style_ref[...]
    gamma = jnp.dot(st, a3w1_ref[...].T, preferred_element_type=jnp.float32) + a3b1_ref[...]
    beta = jnp.dot(st, a3w2_ref[...].T, preferred_element_type=jnp.float32) + a3b2_ref[...]
    out_ref[...] = gamma * (h2 - rm) / (rs + 1e-8) + beta


def _tc3(agg, hp_pad, a_src, a_dst, btab, gat_b, style,
         ad3_w1, ad3_b1, ad3_w2, ad3_b2):
    nb = N_OUT // BLK
    return pl.pallas_call(
        _tc3_body,
        grid=(nb,),
        in_specs=[
            pl.BlockSpec((BLK, DP), lambda i: (i, 0)),
            pl.BlockSpec((BLK, DP), lambda i: (i + N_OUT // BLK, 0)),
            pl.BlockSpec((BLK, DP), lambda i: (i, 0)),
            pl.BlockSpec((BLK,), lambda i: (i,)),
            pl.BlockSpec((BLK,), lambda i: (i,)),
            pl.BlockSpec((BLK,), lambda i: (i,)),
            pl.BlockSpec((D,), lambda i: (0,)),
            pl.BlockSpec((BLK, D), lambda i: (i, 0)),
            pl.BlockSpec((D, D), lambda i: (0, 0)),
            pl.BlockSpec((D,), lambda i: (0,)),
            pl.BlockSpec((D, D), lambda i: (0, 0)),
            pl.BlockSpec((D,), lambda i: (0,)),
        ],
        out_specs=pl.BlockSpec((BLK, D), lambda i: (i, 0)),
        out_shape=jax.ShapeDtypeStruct((N_OUT, D), jnp.float32),
    )(agg, agg, hp_pad, a_src, a_dst, btab, gat_b, style,
      ad3_w1, ad3_b1, ad3_w2, ad3_b2)


def kernel(x, edge_index, style, trs_w, trs_b, bn2_w, bn2_b, ad1_w1, ad1_b1,
           ad1_w2, ad1_b2, fc1_w, fc1_b, bn1_w, bn1_b, ad2_w1, ad2_b1,
           ad2_w2, ad2_b2, gat_w, gat_att_src, gat_att_dst, gat_b, ad3_w1,
           ad3_b1, ad3_w2, ad3_b2):
    u, s1, s2 = _tc1(x, trs_w, trs_b, bn2_w, bn2_b, style, ad1_w1, ad1_b1,
                     ad1_w2, ad1_b2, fc1_w, fc1_b)
    mu = s1 / N_OUT
    var = s2 / N_OUT - mu * mu
    hp_pad, a_src, a_dst, amax = _tc2(u, mu, var, bn1_w, bn1_b, style, ad2_w1,
                                      ad2_b1, ad2_w2, ad2_b2, gat_w,
                                      gat_att_src, gat_att_dst)
    max_a = amax[0, 0]
    tb = max_a + a_dst
    btab = jnp.where(tb >= 0, tb, 0.2 * tb)
    src2d = edge_index[0].reshape(E // 128, 128)
    dst2d = edge_index[1].reshape(E // 128, 128)
    agg = _sc_edge(src2d, dst2d, a_src, a_dst, amax[0, :16], hp_pad)
    return _tc3(agg, hp_pad, a_src, a_dst, btab, gat_b, style,
                ad3_w1, ad3_b1, ad3_w2, ad3_b2)


# speedup vs baseline: 81.9867x; 1.3715x over previous
import jax
import jax.numpy as jnp
from jax import lax
from jax.experimental import pallas as pl
from jax.experimental.pallas import tpu as pltpu
from jax.experimental.pallas import tpu_sc as plsc

N_IN, N_OUT, D, H, E = 2048, 8192, 64, 256, 524288
BLK = 512
DP = 128
NC, NS = 2, 16
NW = NC * NS
EPT = E // NW
W = 128
NWIN = EPT // W
NPAIR = NWIN // 2
RSUB = N_OUT // NS


def _leaky(x):
    return jnp.where(x >= 0, x, 0.2 * x)


def _tc1_body(x_ref, trsw_ref, trsb_ref, bn2w_ref, bn2b_ref, style_ref,
              a1w1_ref, a1b1_ref, a1w2_ref, a1b2_ref, fcw_ref, fcb_ref,
              u_ref, s1_ref, s2_ref):
    y = jnp.dot(trsw_ref[...], x_ref[...], preferred_element_type=jnp.float32)
    y = y + trsb_ref[...][:, None]
    m = jnp.mean(y, axis=1, keepdims=True)
    v = jnp.mean((y - m) ** 2, axis=1, keepdims=True)
    h = (y - m) / jnp.sqrt(v + 1e-5) * bn2w_ref[...][:, None] + bn2b_ref[...][:, None]
    h = _leaky(h)
    rm = jnp.mean(h, axis=1, keepdims=True)
    rs = jnp.sqrt(jnp.sum((h - rm) ** 2, axis=1, keepdims=True) / (D - 1))
    st = style_ref[...]
    gamma = jnp.dot(st, a1w1_ref[...].T, preferred_element_type=jnp.float32) + a1b1_ref[...]
    beta = jnp.dot(st, a1w2_ref[...].T, preferred_element_type=jnp.float32) + a1b2_ref[...]
    h0 = gamma * (h - rm) / (rs + 1e-8) + beta
    u = jnp.dot(h0, fcw_ref[...].T, preferred_element_type=jnp.float32) + fcb_ref[...]
    u_ref[...] = u
    ps1 = jnp.sum(u, axis=0, keepdims=True)
    ps2 = jnp.sum(u * u, axis=0, keepdims=True)

    @pl.when(pl.program_id(0) == 0)
    def _():
        s1_ref[...] = ps1
        s2_ref[...] = ps2

    @pl.when(pl.program_id(0) != 0)
    def _():
        s1_ref[...] = s1_ref[...] + ps1
        s2_ref[...] = s2_ref[...] + ps2


def _tc1(x, trs_w, trs_b, bn2_w, bn2_b, style, ad1_w1, ad1_b1, ad1_w2, ad1_b2,
         fc1_w, fc1_b):
    return pl.pallas_call(
        _tc1_body,
        grid=(N_OUT // BLK,),
        in_specs=[
            pl.BlockSpec((N_IN, D), lambda i: (0, 0)),
            pl.BlockSpec((BLK, N_IN), lambda i: (i, 0)),
            pl.BlockSpec((BLK,), lambda i: (i,)),
            pl.BlockSpec((BLK,), lambda i: (i,)),
            pl.BlockSpec((BLK,), lambda i: (i,)),
            pl.BlockSpec((BLK, D), lambda i: (i, 0)),
            pl.BlockSpec((D, D), lambda i: (0, 0)),
            pl.BlockSpec((D,), lambda i: (0,)),
            pl.BlockSpec((D, D), lambda i: (0, 0)),
            pl.BlockSpec((D,), lambda i: (0,)),
            pl.BlockSpec((H, D), lambda i: (0, 0)),
            pl.BlockSpec((H,), lambda i: (0,)),
        ],
        out_specs=[
            pl.BlockSpec((BLK, H), lambda i: (i, 0)),
            pl.BlockSpec((1, H), lambda i: (0, 0)),
            pl.BlockSpec((1, H), lambda i: (0, 0)),
        ],
        out_shape=[
            jax.ShapeDtypeStruct((N_OUT, H), jnp.float32),
            jax.ShapeDtypeStruct((1, H), jnp.float32),
            jax.ShapeDtypeStruct((1, H), jnp.float32),
        ],
    )(x, trs_w, trs_b, bn2_w, bn2_b, style, ad1_w1, ad1_b1, ad1_w2, ad1_b2,
      fc1_w, fc1_b)


def _tc2_body(u_ref, mu_ref, var_ref, bn1w_ref, bn1b_ref, style_ref,
              a2w1_ref, a2b1_ref, a2w2_ref, a2b2_ref, gatw_ref, asv_ref,
              adv_ref, hp_ref, asrc_ref, adst_ref, amax_ref):
    u = u_ref[...]
    h1 = (u - mu_ref[...]) / jnp.sqrt(var_ref[...] + 1e-5) * bn1w_ref[...] + bn1b_ref[...]
    h1 = _leaky(h1)
    rm = jnp.mean(h1, axis=1, keepdims=True)
    rs = jnp.sqrt(jnp.sum((h1 - rm) ** 2, axis=1, keepdims=True) / (H - 1))
    st = style_ref[...]
    gamma = jnp.dot(st, a2w1_ref[...].T, preferred_element_type=jnp.float32) + a2b1_ref[...]
    beta = jnp.dot(st, a2w2_ref[...].T, preferred_element_type=jnp.float32) + a2b2_ref[...]
    h1n = gamma * (h1 - rm) / (rs + 1e-8) + beta
    hp = jnp.dot(h1n, gatw_ref[...].T, preferred_element_type=jnp.float32)
    one = jnp.ones((hp.shape[0], 1), jnp.float32)
    zer = jnp.zeros((hp.shape[0], DP - D - 1), jnp.float32)
    hp_ref[...] = jnp.concatenate([hp, one, zer], axis=1)
    asrc = jnp.sum(hp * asv_ref[...][None, :], axis=1)
    adst = jnp.sum(hp * adv_ref[...][None, :], axis=1)
    asrc_ref[...] = asrc
    adst_ref[...] = adst
    bm = jnp.max(asrc)

    @pl.when(pl.program_id(0) == 0)
    def _():
        amax_ref[...] = jnp.full((1, 128), bm, jnp.float32)

    @pl.when(pl.program_id(0) != 0)
    def _():
        amax_ref[...] = jnp.maximum(amax_ref[...], bm)


def _tc2(u, mu, var, bn1_w, bn1_b, style, ad2_w1, ad2_b1, ad2_w2, ad2_b2,
         gat_w, att_src, att_dst):
    return pl.pallas_call(
        _tc2_body,
        grid=(N_OUT // BLK,),
        in_specs=[
            pl.BlockSpec((BLK, H), lambda i: (i, 0)),
            pl.BlockSpec((1, H), lambda i: (0, 0)),
            pl.BlockSpec((1, H), lambda i: (0, 0)),
            pl.BlockSpec((H,), lambda i: (0,)),
            pl.BlockSpec((H,), lambda i: (0,)),
            pl.BlockSpec((BLK, D), lambda i: (i, 0)),
            pl.BlockSpec((H, D), lambda i: (0, 0)),
            pl.BlockSpec((H,), lambda i: (0,)),
            pl.BlockSpec((H, D), lambda i: (0, 0)),
            pl.BlockSpec((H,), lambda i: (0,)),
            pl.BlockSpec((D, H), lambda i: (0, 0)),
            pl.BlockSpec((D,), lambda i: (0,)),
            pl.BlockSpec((D,), lambda i: (0,)),
        ],
        out_specs=[
            pl.BlockSpec((BLK, DP), lambda i: (i, 0)),
            pl.BlockSpec((BLK,), lambda i: (i,)),
            pl.BlockSpec((BLK,), lambda i: (i,)),
            pl.BlockSpec((1, 128), lambda i: (0, 0)),
        ],
        out_shape=[
            jax.ShapeDtypeStruct((N_OUT, DP), jnp.float32),
            jax.ShapeDtypeStruct((N_OUT,), jnp.float32),
            jax.ShapeDtypeStruct((N_OUT,), jnp.float32),
            jax.ShapeDtypeStruct((1, 128), jnp.float32),
        ],
    )(u, mu, var, bn1_w, bn1_b, style, ad2_w1, ad2_b1, ad2_w2, ad2_b2,
      gat_w, att_src, att_dst)


def _sc_edge_body(src_hbm, dst_hbm, asrc_hbm, adst_hbm, amax_hbm, hp_hbm,
                  agg_out,
                  src_w, dst_w, dsts_w, asrc_t, adst_t, amax_t, rows_a, rows_b,
                  agg_sh, sg_a, sg_b, ss_a, ss_b, si0, si1):
    cid = lax.axis_index("c")
    sid = lax.axis_index("s")
    wid = cid * NS + sid
    rowbase = wid * NWIN
    z16 = jnp.zeros((16,), jnp.float32)

    pltpu.sync_copy(asrc_hbm, asrc_t)
    pltpu.sync_copy(adst_hbm, adst_t)
    pltpu.sync_copy(amax_hbm, amax_t)

    def _zrow(j, carry):
        for q in range(DP // 16):
            rows_a[j, pl.ds(q * 16, 16)] = z16
        return carry
    lax.fori_loop(0, W, _zrow, 0)

    for k in range(RSUB // W):
        pltpu.sync_copy(rows_a, agg_sh.at[pl.ds(sid * RSUB + k * W, W)])
    plsc.subcore_barrier()
    av = amax_t[...]

    rows = (rows_a, rows_b)
    sg = (sg_a, sg_b)
    ss = (ss_a, ss_b)
    si = (si0, si1)

    def _idx_issue(slot, row, sem):
        pltpu.async_copy(src_hbm.at[pl.ds(row, 1)],
                         src_w.at[pl.ds(slot, 1)], sem)
        pltpu.async_copy(dst_hbm.at[pl.ds(row, 1)],
                         dst_w.at[pl.ds(slot, 1)], sem)

    def _idx_wait(slot, row, sem):
        pltpu.make_async_copy(src_hbm.at[pl.ds(row, 1)],
                              src_w.at[pl.ds(slot, 1)], sem).wait()
        pltpu.make_async_copy(dst_hbm.at[pl.ds(row, 1)],
                              dst_w.at[pl.ds(slot, 1)], sem).wait()

    def _compute(slot):
        rv = rows[slot]

        def _vreg(k2, carry):
            o = pl.multiple_of(k2 * 16, 16)
            idx_s = src_w[slot, pl.ds(o, 16)]
            idx_d = dst_w[slot, pl.ds(o, 16)]
            dsts_w[slot, pl.ds(o, 16)] = idx_d
            s = plsc.load_gather(asrc_t, [idx_s])
            t = plsc.load_gather(adst_t, [idx_d])
            stv = s + t
            e = jnp.where(stv >= 0, stv, 0.2 * stv)
            bv = av + t
            b = jnp.where(bv >= 0, bv, 0.2 * bv)
            ex = jnp.exp(e - b)
            for l in range(16):
                wv = jnp.full((16,), ex[l], jnp.float32)
                for q in range(5):
                    rv[o + l, pl.ds(q * 16, 16)] = (
                        rv[o + l, pl.ds(q * 16, 16)] * wv)
            return carry
        lax.fori_loop(0, W // 16, _vreg, 0)

    def _pair(t, first, last):
        r_a = rowbase + 2 * t
        _idx_wait(1, r_a + 1, si[1])
        if not first:
            pltpu.make_async_copy(rows[1], agg_sh.at[dsts_w.at[1]],
                                  ss[1]).wait()
        pltpu.async_copy(hp_hbm.at[src_w.at[1]], rows[1], sg[1])
        pltpu.make_async_copy(hp_hbm.at[src_w.at[0]], rows[0], sg[0]).wait()
        _compute(0)
        pltpu.async_copy(rows[0], agg_sh.at[dsts_w.at[0]], ss[0], add=True)
        if not last:
            _idx_issue(0, r_a + 2, si[0])
        pltpu.make_async_copy(hp_hbm.at[src_w.at[1]], rows[1], sg[1]).wait()
        _compute(1)
        pltpu.async_copy(rows[1], agg_sh.at[dsts_w.at[1]], ss[1], add=True)
        if not last:
            _idx_issue(1, r_a + 3, si[1])
        pltpu.make_async_copy(rows[0], agg_sh.at[dsts_w.at[0]], ss[0]).wait()
        if not last:
            _idx_wait(0, r_a + 2, si[0])
            pltpu.async_copy(hp_hbm.at[src_w.at[0]], rows[0], sg[0])

    pltpu.sync_copy(src_hbm.at[pl.ds(rowbase, 1)], src_w.at[pl.ds(0, 1)])
    pltpu.sync_copy(dst_hbm.at[pl.ds(rowbase, 1)], dst_w.at[pl.ds(0, 1)])
    pltpu.async_copy(hp_hbm.at[src_w.at[0]], rows[0], sg[0])
    _idx_issue(1, rowbase + 1, si[1])

    _pair(0, True, False)

    def _body(t, carry):
        _pair(t, False, False)
        return carry
    lax.fori_loop(1, NPAIR - 1, _body, 0)

    _pair(NPAIR - 1, False, True)
    pltpu.make_async_copy(rows[1], agg_sh.at[dsts_w.at[1]], ss[1]).wait()

    plsc.subcore_barrier()

    out_base = cid * N_OUT + sid * RSUB
    pltpu.sync_copy(agg_sh.at[pl.ds(sid * RSUB, RSUB)],
                    agg_out.at[pl.ds(out_base, RSUB)])


def _sc_edge(src2d, dst2d, a_src, a_dst, amax16, hp_pad):
    mesh = plsc.VectorSubcoreMesh(core_axis_name="c", subcore_axis_name="s",
                                  num_cores=NC, num_subcores=NS)
    return pl.kernel(
        _sc_edge_body,
        out_type=jax.ShapeDtypeStruct((NC * N_OUT, DP), jnp.float32),
        mesh=mesh,
        compiler_params=pltpu.CompilerParams(needs_layout_passes=False),
        scratch_types=[
            pltpu.VMEM((2, 128), jnp.int32),
            pltpu.VMEM((2, 128), jnp.int32),
            pltpu.VMEM((2, 128), jnp.int32),
            pltpu.VMEM((N_OUT,), jnp.float32),
            pltpu.VMEM((N_OUT,), jnp.float32),
            pltpu.VMEM((16,), jnp.float32),
            pltpu.VMEM((W, DP), jnp.float32),
            pltpu.VMEM((W, DP), jnp.float32),
            pltpu.VMEM_SHARED((N_OUT, DP), jnp.float32),
            pltpu.SemaphoreType.DMA,
            pltpu.SemaphoreType.DMA,
            pltpu.SemaphoreType.DMA,
            pltpu.SemaphoreType.DMA,
            pltpu.SemaphoreType.DMA,
            pltpu.SemaphoreType.DMA,
        ],
    )(src2d, dst2d, a_src, a_dst, amax16, hp_pad)


def _tc3_body(agg0_ref, agg1_ref, hp_ref, asrc_ref, adst_ref, btab_ref,
              gatb_ref, style_ref, a3w1_ref, a3b1_ref, a3w2_ref, a3b2_ref,
              out_ref):
    es = _leaky(asrc_ref[...] + adst_ref[...])
    ex_self = jnp.exp(es - btab_ref[...])
    hp = hp_ref[...][:, :D]
    den = agg0_ref[...][:, D] + agg1_ref[...][:, D] + ex_self
    aggr = agg0_ref[...][:, :D] + agg1_ref[...][:, :D] + ex_self[:, None] * hp
    agg = aggr / (den[:, None] + 1e-16) + gatb_ref[...]
    h2 = _leaky(agg)
    rm = jnp.mean(h2, axis=1, keepdims=True)
    rs = jnp.sqrt(jnp.sum((h2 - rm) ** 2, axis=1, keepdims=True) / (D - 1))
    st = style_ref[...]
    gamma = jnp.dot(st, a3w1_ref[...].T, preferred_element_type=jnp.float32) + a3b1_ref[...]
    beta = jnp.dot(st, a3w2_ref[...].T, preferred_element_type=jnp.float32) + a3b2_ref[...]
    out_ref[...] = gamma * (h2 - rm) / (rs + 1e-8) + beta


def _tc3(agg, hp_pad, a_src, a_dst, btab, gat_b, style,
         ad3_w1, ad3_b1, ad3_w2, ad3_b2):
    nb = N_OUT // BLK
    return pl.pallas_call(
        _tc3_body,
        grid=(nb,),
        in_specs=[
            pl.BlockSpec((BLK, DP), lambda i: (i, 0)),
            pl.BlockSpec((BLK, DP), lambda i: (i + N_OUT // BLK, 0)),
            pl.BlockSpec((BLK, DP), lambda i: (i, 0)),
            pl.BlockSpec((BLK,), lambda i: (i,)),
            pl.BlockSpec((BLK,), lambda i: (i,)),
            pl.BlockSpec((BLK,), lambda i: (i,)),
            pl.BlockSpec((D,), lambda i: (0,)),
            pl.BlockSpec((BLK, D), lambda i: (i, 0)),
            pl.BlockSpec((D, D), lambda i: (0, 0)),
            pl.BlockSpec((D,), lambda i: (0,)),
            pl.BlockSpec((D, D), lambda i: (0, 0)),
            pl.BlockSpec((D,), lambda i: (0,)),
        ],
        out_specs=pl.BlockSpec((BLK, D), lambda i: (i, 0)),
        out_shape=jax.ShapeDtypeStruct((N_OUT, D), jnp.float32),
    )(agg, agg, hp_pad, a_src, a_dst, btab, gat_b, style,
      ad3_w1, ad3_b1, ad3_w2, ad3_b2)


def kernel(x, edge_index, style, trs_w, trs_b, bn2_w, bn2_b, ad1_w1, ad1_b1,
           ad1_w2, ad1_b2, fc1_w, fc1_b, bn1_w, bn1_b, ad2_w1, ad2_b1,
           ad2_w2, ad2_b2, gat_w, gat_att_src, gat_att_dst, gat_b, ad3_w1,
           ad3_b1, ad3_w2, ad3_b2):
    u, s1, s2 = _tc1(x, trs_w, trs_b, bn2_w, bn2_b, style, ad1_w1, ad1_b1,
                     ad1_w2, ad1_b2, fc1_w, fc1_b)
    mu = s1 / N_OUT
    var = s2 / N_OUT - mu * mu
    hp_pad, a_src, a_dst, amax = _tc2(u, mu, var, bn1_w, bn1_b, style, ad2_w1,
                                      ad2_b1, ad2_w2, ad2_b2, gat_w,
                                      gat_att_src, gat_att_dst)
    max_a = amax[0, 0]
    tb = max_a + a_dst
    btab = jnp.where(tb >= 0, tb, 0.2 * tb)
    src2d = edge_index[0].reshape(E // 128, 128)
    dst2d = edge_index[1].reshape(E // 128, 128)
    agg = _sc_edge(src2d, dst2d, a_src, a_dst, amax[0, :16], hp_pad)
    return _tc3(agg, hp_pad, a_src, a_dst, btab, gat_b, style,
                ad3_w1, ad3_b1, ad3_w2, ad3_b2)


# speedup vs baseline: 93.2866x; 1.1378x over previous
import jax
import jax.numpy as jnp
from jax import lax
from jax.experimental import pallas as pl
from jax.experimental.pallas import tpu as pltpu
from jax.experimental.pallas import tpu_sc as plsc

N_IN, N_OUT, D, H, E = 2048, 8192, 64, 256, 524288
BLK = 512
DP = 128
NC, NS = 2, 16
NW = NC * NS
EPT = E // NW
W = 128
NWIN = EPT // W
NTRI = (NWIN - 2) // 3
RSUB = N_OUT // NS


def _leaky(x):
    return jnp.where(x >= 0, x, 0.2 * x)


def _tc1_body(x_ref, trsw_ref, trsb_ref, bn2w_ref, bn2b_ref, style_ref,
              a1w1_ref, a1b1_ref, a1w2_ref, a1b2_ref, fcw_ref, fcb_ref,
              u_ref, s1_ref, s2_ref):
    y = jnp.dot(trsw_ref[...], x_ref[...], preferred_element_type=jnp.float32)
    y = y + trsb_ref[...][:, None]
    m = jnp.mean(y, axis=1, keepdims=True)
    v = jnp.mean((y - m) ** 2, axis=1, keepdims=True)
    h = (y - m) / jnp.sqrt(v + 1e-5) * bn2w_ref[...][:, None] + bn2b_ref[...][:, None]
    h = _leaky(h)
    rm = jnp.mean(h, axis=1, keepdims=True)
    rs = jnp.sqrt(jnp.sum((h - rm) ** 2, axis=1, keepdims=True) / (D - 1))
    st = style_ref[...]
    gamma = jnp.dot(st, a1w1_ref[...].T, preferred_element_type=jnp.float32) + a1b1_ref[...]
    beta = jnp.dot(st, a1w2_ref[...].T, preferred_element_type=jnp.float32) + a1b2_ref[...]
    h0 = gamma * (h - rm) / (rs + 1e-8) + beta
    u = jnp.dot(h0, fcw_ref[...].T, preferred_element_type=jnp.float32) + fcb_ref[...]
    u_ref[...] = u
    ps1 = jnp.sum(u, axis=0, keepdims=True)
    ps2 = jnp.sum(u * u, axis=0, keepdims=True)

    @pl.when(pl.program_id(0) == 0)
    def _():
        s1_ref[...] = ps1
        s2_ref[...] = ps2

    @pl.when(pl.program_id(0) != 0)
    def _():
        s1_ref[...] = s1_ref[...] + ps1
        s2_ref[...] = s2_ref[...] + ps2


def _tc1(x, trs_w, trs_b, bn2_w, bn2_b, style, ad1_w1, ad1_b1, ad1_w2, ad1_b2,
         fc1_w, fc1_b):
    return pl.pallas_call(
        _tc1_body,
        grid=(N_OUT // BLK,),
        in_specs=[
            pl.BlockSpec((N_IN, D), lambda i: (0, 0)),
            pl.BlockSpec((BLK, N_IN), lambda i: (i, 0)),
            pl.BlockSpec((BLK,), lambda i: (i,)),
            pl.BlockSpec((BLK,), lambda i: (i,)),
            pl.BlockSpec((BLK,), lambda i: (i,)),
            pl.BlockSpec((BLK, D), lambda i: (i, 0)),
            pl.BlockSpec((D, D), lambda i: (0, 0)),
            pl.BlockSpec((D,), lambda i: (0,)),
            pl.BlockSpec((D, D), lambda i: (0, 0)),
            pl.BlockSpec((D,), lambda i: (0,)),
            pl.BlockSpec((H, D), lambda i: (0, 0)),
            pl.BlockSpec((H,), lambda i: (0,)),
        ],
        out_specs=[
            pl.BlockSpec((BLK, H), lambda i: (i, 0)),
            pl.BlockSpec((1, H), lambda i: (0, 0)),
            pl.BlockSpec((1, H), lambda i: (0, 0)),
        ],
        out_shape=[
            jax.ShapeDtypeStruct((N_OUT, H), jnp.float32),
            jax.ShapeDtypeStruct((1, H), jnp.float32),
            jax.ShapeDtypeStruct((1, H), jnp.float32),
        ],
    )(x, trs_w, trs_b, bn2_w, bn2_b, style, ad1_w1, ad1_b1, ad1_w2, ad1_b2,
      fc1_w, fc1_b)


def _tc2_body(u_ref, mu_ref, var_ref, bn1w_ref, bn1b_ref, style_ref,
              a2w1_ref, a2b1_ref, a2w2_ref, a2b2_ref, gatw_ref, asv_ref,
              adv_ref, hp_ref, asrc_ref, adst_ref, amax_ref):
    u = u_ref[...]
    h1 = (u - mu_ref[...]) / jnp.sqrt(var_ref[...] + 1e-5) * bn1w_ref[...] + bn1b_ref[...]
    h1 = _leaky(h1)
    rm = jnp.mean(h1, axis=1, keepdims=True)
    rs = jnp.sqrt(jnp.sum((h1 - rm) ** 2, axis=1, keepdims=True) / (H - 1))
    st = style_ref[...]
    gamma = jnp.dot(st, a2w1_ref[...].T, preferred_element_type=jnp.float32) + a2b1_ref[...]
    beta = jnp.dot(st, a2w2_ref[...].T, preferred_element_type=jnp.float32) + a2b2_ref[...]
    h1n = gamma * (h1 - rm) / (rs + 1e-8) + beta
    hp = jnp.dot(h1n, gatw_ref[...].T, preferred_element_type=jnp.float32)
    asrc = jnp.sum(hp * asv_ref[...][None, :], axis=1)
    adst = jnp.sum(hp * adv_ref[...][None, :], axis=1)
    one = jnp.ones((hp.shape[0], 1), jnp.float32)
    zer = jnp.zeros((hp.shape[0], DP - D - 2), jnp.float32)
    hp_ref[...] = jnp.concatenate([hp, one, asrc[:, None], zer], axis=1)
    asrc_ref[...] = asrc
    adst_ref[...] = adst
    bm = jnp.max(asrc)

    @pl.when(pl.program_id(0) == 0)
    def _():
        amax_ref[...] = jnp.full((1, 128), bm, jnp.float32)

    @pl.when(pl.program_id(0) != 0)
    def _():
        amax_ref[...] = jnp.maximum(amax_ref[...], bm)


def _tc2(u, mu, var, bn1_w, bn1_b, style, ad2_w1, ad2_b1, ad2_w2, ad2_b2,
         gat_w, att_src, att_dst):
    return pl.pallas_call(
        _tc2_body,
        grid=(N_OUT // BLK,),
        in_specs=[
            pl.BlockSpec((BLK, H), lambda i: (i, 0)),
            pl.BlockSpec((1, H), lambda i: (0, 0)),
            pl.BlockSpec((1, H), lambda i: (0, 0)),
            pl.BlockSpec((H,), lambda i: (0,)),
            pl.BlockSpec((H,), lambda i: (0,)),
            pl.BlockSpec((BLK, D), lambda i: (i, 0)),
            pl.BlockSpec((H, D), lambda i: (0, 0)),
            pl.BlockSpec((H,), lambda i: (0,)),
            pl.BlockSpec((H, D), lambda i: (0, 0)),
            pl.BlockSpec((H,), lambda i: (0,)),
            pl.BlockSpec((D, H), lambda i: (0, 0)),
            pl.BlockSpec((D,), lambda i: (0,)),
            pl.BlockSpec((D,), lambda i: (0,)),
        ],
        out_specs=[
            pl.BlockSpec((BLK, DP), lambda i: (i, 0)),
            pl.BlockSpec((BLK,), lambda i: (i,)),
            pl.BlockSpec((BLK,), lambda i: (i,)),
            pl.BlockSpec((1, 128), lambda i: (0, 0)),
        ],
        out_shape=[
            jax.ShapeDtypeStruct((N_OUT, DP), jnp.float32),
            jax.ShapeDtypeStruct((N_OUT,), jnp.float32),
            jax.ShapeDtypeStruct((N_OUT,), jnp.float32),
            jax.ShapeDtypeStruct((1, 128), jnp.float32),
        ],
    )(u, mu, var, bn1_w, bn1_b, style, ad2_w1, ad2_b1, ad2_w2, ad2_b2,
      gat_w, att_src, att_dst)


def _sc_edge_body(src_hbm, dst_hbm, adst_hbm, amax_hbm, hp_hbm,
                  agg_out,
                  src_w, dst_w, dsts_w, adst_t, amax_t, rows_a, rows_b, rows_c,
                  agg_sh, sg0, sg1, sg2, ss0, ss1, ss2, si0, si1, si2):
    cid = lax.axis_index("c")
    sid = lax.axis_index("s")
    wid = cid * NS + sid
    rowbase = wid * NWIN
    z16 = jnp.zeros((16,), jnp.float32)

    pltpu.sync_copy(adst_hbm, adst_t)
    pltpu.sync_copy(amax_hbm, amax_t)

    def _zrow(j, carry):
        for q in range(DP // 16):
            rows_a[j, pl.ds(q * 16, 16)] = z16
        return carry
    lax.fori_loop(0, W, _zrow, 0)

    for k in range(RSUB // W):
        pltpu.sync_copy(rows_a, agg_sh.at[pl.ds(sid * RSUB + k * W, W)])
    plsc.subcore_barrier()
    av = amax_t[...]

    rows = (rows_a, rows_b, rows_c)
    sg = (sg0, sg1, sg2)
    ss = (ss0, ss1, ss2)
    si = (si0, si1, si2)

    def _idx_issue(slot, row, sem):
        pltpu.async_copy(src_hbm.at[pl.ds(row, 1)],
                         src_w.at[pl.ds(slot, 1)], sem)
        pltpu.async_copy(dst_hbm.at[pl.ds(row, 1)],
                         dst_w.at[pl.ds(slot, 1)], sem)

    def _idx_wait(slot, row, sem):
        pltpu.make_async_copy(src_hbm.at[pl.ds(row, 1)],
                              src_w.at[pl.ds(slot, 1)], sem).wait()
        pltpu.make_async_copy(dst_hbm.at[pl.ds(row, 1)],
                              dst_w.at[pl.ds(slot, 1)], sem).wait()

    def _gath_issue(slot):
        pltpu.async_copy(hp_hbm.at[src_w.at[slot]], rows[slot], sg[slot])

    def _gath_wait(slot):
        pltpu.make_async_copy(hp_hbm.at[src_w.at[slot]], rows[slot],
                              sg[slot]).wait()

    def _scat_issue(slot):
        pltpu.async_copy(rows[slot], agg_sh.at[dsts_w.at[slot]], ss[slot],
                         add=True)

    def _scat_wait(slot):
        pltpu.make_async_copy(rows[slot], agg_sh.at[dsts_w.at[slot]],
                              ss[slot]).wait()

    def _compute(slot):
        rv = rows[slot]
        c65 = jnp.full((16,), D + 1, jnp.int32)

        def _vreg(k2, carry):
            o = pl.multiple_of(k2 * 16, 16)
            idx_d = dst_w[slot, pl.ds(o, 16)]
            dsts_w[slot, pl.ds(o, 16)] = idx_d
            ridx = lax.iota(jnp.int32, 16) + o
            s = plsc.load_gather(rv, [ridx, c65])
            t = plsc.load_gather(adst_t, [idx_d])
            stv = s + t
            e = jnp.where(stv >= 0, stv, 0.2 * stv)
            bv = av + t
            b = jnp.where(bv >= 0, bv, 0.2 * bv)
            ex = jnp.exp(e - b)
            for l in range(16):
                wv = jnp.full((16,), ex[l], jnp.float32)
                for q in range(5):
                    rv[o + l, pl.ds(q * 16, 16)] = (
                        rv[o + l, pl.ds(q * 16, 16)] * wv)
            return carry
        lax.fori_loop(0, W // 16, _vreg, 0)

    def _tri(t, first, last):
        r0 = rowbase + 3 * t
        _idx_wait(1, r0 + 1, si[1])
        if not first:
            _scat_wait(1)
        _gath_issue(1)
        _gath_wait(0)
        _compute(0)
        _scat_issue(0)
        if not last:
            _idx_issue(0, r0 + 3, si[0])
        _idx_wait(2, r0 + 2, si[2])
        if not first:
            _scat_wait(2)
        _gath_issue(2)
        _gath_wait(1)
        _compute(1)
        _scat_issue(1)
        if not last:
            _idx_issue(1, r0 + 4, si[1])
        _scat_wait(0)
        if not last:
            _idx_wait(0, r0 + 3, si[0])
            _gath_issue(0)
        _gath_wait(2)
        _compute(2)
        _scat_issue(2)
        if not last:
            _idx_issue(2, r0 + 5, si[2])

    pltpu.sync_copy(src_hbm.at[pl.ds(rowbase, 1)], src_w.at[pl.ds(0, 1)])
    pltpu.sync_copy(dst_hbm.at[pl.ds(rowbase, 1)], dst_w.at[pl.ds(0, 1)])
    _gath_issue(0)
    _idx_issue(1, rowbase + 1, si[1])
    _idx_issue(2, rowbase + 2, si[2])

    _tri(0, True, False)

    def _body(t, carry):
        _tri(t, False, False)
        return carry
    lax.fori_loop(1, NTRI - 1, _body, 0)

    _tri(NTRI - 1, False, True)

    tb = rowbase + 3 * NTRI
    pltpu.sync_copy(src_hbm.at[pl.ds(tb, 1)], src_w.at[pl.ds(0, 1)])
    pltpu.sync_copy(dst_hbm.at[pl.ds(tb, 1)], dst_w.at[pl.ds(0, 1)])
    _gath_issue(0)
    pltpu.sync_copy(src_hbm.at[pl.ds(tb + 1, 1)], src_w.at[pl.ds(1, 1)])
    pltpu.sync_copy(dst_hbm.at[pl.ds(tb + 1, 1)], dst_w.at[pl.ds(1, 1)])
    _scat_wait(1)
    _gath_issue(1)
    _gath_wait(0)
    _compute(0)
    _scat_issue(0)
    _gath_wait(1)
    _compute(1)
    _scat_issue(1)
    _scat_wait(0)
    _scat_wait(1)
    _scat_wait(2)

    plsc.subcore_barrier()

    out_base = cid * N_OUT + sid * RSUB
    pltpu.sync_copy(agg_sh.at[pl.ds(sid * RSUB, RSUB)],
                    agg_out.at[pl.ds(out_base, RSUB)])


def _sc_edge(src2d, dst2d, a_dst, amax16, hp_pad):
    mesh = plsc.VectorSubcoreMesh(core_axis_name="c", subcore_axis_name="s",
                                  num_cores=NC, num_subcores=NS)
    return pl.kernel(
        _sc_edge_body,
        out_type=jax.ShapeDtypeStruct((NC * N_OUT, DP), jnp.float32),
        mesh=mesh,
        compiler_params=pltpu.CompilerParams(needs_layout_passes=False),
        scratch_types=[
            pltpu.VMEM((3, 128), jnp.int32),
            pltpu.VMEM((3, 128), jnp.int32),
            pltpu.VMEM((3, 128), jnp.int32),
            pltpu.VMEM((N_OUT,), jnp.float32),
            pltpu.VMEM((16,), jnp.float32),
            pltpu.VMEM((W, DP), jnp.float32),
            pltpu.VMEM((W, DP), jnp.float32),
            pltpu.VMEM((W, DP), jnp.float32),
            pltpu.VMEM_SHARED((N_OUT, DP), jnp.float32),
            pltpu.SemaphoreType.DMA,
            pltpu.SemaphoreType.DMA,
            pltpu.SemaphoreType.DMA,
            pltpu.SemaphoreType.DMA,
            pltpu.SemaphoreType.DMA,
            pltpu.SemaphoreType.DMA,
            pltpu.SemaphoreType.DMA,
            pltpu.SemaphoreType.DMA,
            pltpu.SemaphoreType.DMA,
        ],
    )(src2d, dst2d, a_dst, amax16, hp_pad)


def _tc3_body(agg0_ref, agg1_ref, hp_ref, asrc_ref, adst_ref, btab_ref,
              gatb_ref, style_ref, a3w1_ref, a3b1_ref, a3w2_ref, a3b2_ref,
              out_ref):
    es = _leaky(asrc_ref[...] + adst_ref[...])
    ex_self = jnp.exp(es - btab_ref[...])
    hp = hp_ref[...][:, :D]
    den = agg0_ref[...][:, D] + agg1_ref[...][:, D] + ex_self
    aggr = agg0_ref[...][:, :D] + agg1_ref[...][:, :D] + ex_self[:, None] * hp
    agg = aggr / (den[:, None] + 1e-16) + gatb_ref[...]
    h2 = _leaky(agg)
    rm = jnp.mean(h2, axis=1, keepdims=True)
    rs = jnp.sqrt(jnp.sum((h2 - rm) ** 2, axis=1, keepdims=True) / (D - 1))
    st = style_ref[...]
    gamma = jnp.dot(st, a3w1_ref[...].T, preferred_element_type=jnp.float32) + a3b1_ref[...]
    beta = jnp.dot(st, a3w2_ref[...].T, preferred_element_type=jnp.float32) + a3b2_ref[...]
    out_ref[...] = gamma * (h2 - rm) / (rs + 1e-8) + beta


def _tc3(agg, hp_pad, a_src, a_dst, btab, gat_b, style,
         ad3_w1, ad3_b1, ad3_w2, ad3_b2):
    nb = N_OUT // BLK
    return pl.pallas_call(
        _tc3_body,
        grid=(nb,),
        in_specs=[
            pl.BlockSpec((BLK, DP), lambda i: (i, 0)),
            pl.BlockSpec((BLK, DP), lambda i: (i + N_OUT // BLK, 0)),
            pl.BlockSpec((BLK, DP), lambda i: (i, 0)),
            pl.BlockSpec((BLK,), lambda i: (i,)),
            pl.BlockSpec((BLK,), lambda i: (i,)),
            pl.BlockSpec((BLK,), lambda i: (i,)),
            pl.BlockSpec((D,), lambda i: (0,)),
            pl.BlockSpec((BLK, D), lambda i: (i, 0)),
            pl.BlockSpec((D, D), lambda i: (0, 0)),
            pl.BlockSpec((D,), lambda i: (0,)),
            pl.BlockSpec((D, D), lambda i: (0, 0)),
            pl.BlockSpec((D,), lambda i: (0,)),
        ],
        out_specs=pl.BlockSpec((BLK, D), lambda i: (i, 0)),
        out_shape=jax.ShapeDtypeStruct((N_OUT, D), jnp.float32),
    )(agg, agg, hp_pad, a_src, a_dst, btab, gat_b, style,
      ad3_w1, ad3_b1, ad3_w2, ad3_b2)


def kernel(x, edge_index, style, trs_w, trs_b, bn2_w, bn2_b, ad1_w1, ad1_b1,
           ad1_w2, ad1_b2, fc1_w, fc1_b, bn1_w, bn1_b, ad2_w1, ad2_b1,
           ad2_w2, ad2_b2, gat_w, gat_att_src, gat_att_dst, gat_b, ad3_w1,
           ad3_b1, ad3_w2, ad3_b2):
    u, s1, s2 = _tc1(x, trs_w, trs_b, bn2_w, bn2_b, style, ad1_w1, ad1_b1,
                     ad1_w2, ad1_b2, fc1_w, fc1_b)
    mu = s1 / N_OUT
    var = s2 / N_OUT - mu * mu
    hp_pad, a_src, a_dst, amax = _tc2(u, mu, var, bn1_w, bn1_b, style, ad2_w1,
                                      ad2_b1, ad2_w2, ad2_b2, gat_w,
                                      gat_att_src, gat_att_dst)
    max_a = amax[0, 0]
    tb = max_a + a_dst
    btab = jnp.where(tb >= 0, tb, 0.2 * tb)
    src2d = edge_index[0].reshape(E // 128, 128)
    dst2d = edge_index[1].reshape(E // 128, 128)
    agg = _sc_edge(src2d, dst2d, a_dst, amax[0, :16], hp_pad)
    return _tc3(agg, hp_pad, a_src, a_dst, btab, gat_b, style,
                ad3_w1, ad3_b1, ad3_w2, ad3_b2)


# speedup vs baseline: 94.2188x; 1.0100x over previous
import jax
import jax.numpy as jnp
from jax import lax
from jax.experimental import pallas as pl
from jax.experimental.pallas import tpu as pltpu
from jax.experimental.pallas import tpu_sc as plsc

N_IN, N_OUT, D, H, E = 2048, 8192, 64, 256, 524288
BLK = 512
DP = 128
NC, NS = 2, 16
NW = NC * NS
EPT = E // NW
W = 128
NWIN = EPT // W
NTRI = (NWIN - 2) // 3
RSUB = N_OUT // NS


def _leaky(x):
    return jnp.where(x >= 0, x, 0.2 * x)


def _tc1_body(x_ref, trsw_ref, trsb_ref, bn2w_ref, bn2b_ref, style_ref,
              a1w1_ref, a1b1_ref, a1w2_ref, a1b2_ref, fcw_ref, fcb_ref,
              u_ref, s1_ref, s2_ref):
    y = jnp.dot(trsw_ref[...], x_ref[...], preferred_element_type=jnp.float32)
    y = y + trsb_ref[...][:, None]
    m = jnp.mean(y, axis=1, keepdims=True)
    v = jnp.mean((y - m) ** 2, axis=1, keepdims=True)
    h = (y - m) / jnp.sqrt(v + 1e-5) * bn2w_ref[...][:, None] + bn2b_ref[...][:, None]
    h = _leaky(h)
    rm = jnp.mean(h, axis=1, keepdims=True)
    rs = jnp.sqrt(jnp.sum((h - rm) ** 2, axis=1, keepdims=True) / (D - 1))
    st = style_ref[...]
    gamma = jnp.dot(st, a1w1_ref[...].T, preferred_element_type=jnp.float32) + a1b1_ref[...]
    beta = jnp.dot(st, a1w2_ref[...].T, preferred_element_type=jnp.float32) + a1b2_ref[...]
    h0 = gamma * (h - rm) / (rs + 1e-8) + beta
    u = jnp.dot(h0, fcw_ref[...].T, preferred_element_type=jnp.float32) + fcb_ref[...]
    u_ref[...] = u
    ps1 = jnp.sum(u, axis=0, keepdims=True)
    ps2 = jnp.sum(u * u, axis=0, keepdims=True)

    @pl.when(pl.program_id(0) == 0)
    def _():
        s1_ref[...] = ps1
        s2_ref[...] = ps2

    @pl.when(pl.program_id(0) != 0)
    def _():
        s1_ref[...] = s1_ref[...] + ps1
        s2_ref[...] = s2_ref[...] + ps2


def _tc1(x, trs_w, trs_b, bn2_w, bn2_b, style, ad1_w1, ad1_b1, ad1_w2, ad1_b2,
         fc1_w, fc1_b):
    return pl.pallas_call(
        _tc1_body,
        grid=(N_OUT // BLK,),
        in_specs=[
            pl.BlockSpec((N_IN, D), lambda i: (0, 0)),
            pl.BlockSpec((BLK, N_IN), lambda i: (i, 0)),
            pl.BlockSpec((BLK,), lambda i: (i,)),
            pl.BlockSpec((BLK,), lambda i: (i,)),
            pl.BlockSpec((BLK,), lambda i: (i,)),
            pl.BlockSpec((BLK, D), lambda i: (i, 0)),
            pl.BlockSpec((D, D), lambda i: (0, 0)),
            pl.BlockSpec((D,), lambda i: (0,)),
            pl.BlockSpec((D, D), lambda i: (0, 0)),
            pl.BlockSpec((D,), lambda i: (0,)),
            pl.BlockSpec((H, D), lambda i: (0, 0)),
            pl.BlockSpec((H,), lambda i: (0,)),
        ],
        out_specs=[
            pl.BlockSpec((BLK, H), lambda i: (i, 0)),
            pl.BlockSpec((1, H), lambda i: (0, 0)),
            pl.BlockSpec((1, H), lambda i: (0, 0)),
        ],
        out_shape=[
            jax.ShapeDtypeStruct((N_OUT, H), jnp.float32),
            jax.ShapeDtypeStruct((1, H), jnp.float32),
            jax.ShapeDtypeStruct((1, H), jnp.float32),
        ],
    )(x, trs_w, trs_b, bn2_w, bn2_b, style, ad1_w1, ad1_b1, ad1_w2, ad1_b2,
      fc1_w, fc1_b)


def _tc2_body(u_ref, s1_ref, s2_ref, bn1w_ref, bn1b_ref, style_ref,
              a2w1_ref, a2b1_ref, a2w2_ref, a2b2_ref, gatw_ref, asv_ref,
              adv_ref, hp_ref, asrc_ref, adst_ref, amax_ref):
    u = u_ref[...]
    mu = s1_ref[...] * (1.0 / N_OUT)
    var = s2_ref[...] * (1.0 / N_OUT) - mu * mu
    h1 = (u - mu) / jnp.sqrt(var + 1e-5) * bn1w_ref[...] + bn1b_ref[...]
    h1 = _leaky(h1)
    rm = jnp.mean(h1, axis=1, keepdims=True)
    rs = jnp.sqrt(jnp.sum((h1 - rm) ** 2, axis=1, keepdims=True) / (H - 1))
    st = style_ref[...]
    gamma = jnp.dot(st, a2w1_ref[...].T, preferred_element_type=jnp.float32) + a2b1_ref[...]
    beta = jnp.dot(st, a2w2_ref[...].T, preferred_element_type=jnp.float32) + a2b2_ref[...]
    h1n = gamma * (h1 - rm) / (rs + 1e-8) + beta
    hp = jnp.dot(h1n, gatw_ref[...].T, preferred_element_type=jnp.float32)
    asrc = jnp.sum(hp * asv_ref[...][None, :], axis=1)
    adst = jnp.sum(hp * adv_ref[...][None, :], axis=1)
    one = jnp.ones((hp.shape[0], 1), jnp.float32)
    zer = jnp.zeros((hp.shape[0], DP - D - 2), jnp.float32)
    hp_ref[...] = jnp.concatenate([hp, one, asrc[:, None], zer], axis=1)
    asrc_ref[...] = asrc
    adst_ref[...] = adst
    bm = jnp.max(asrc)

    @pl.when(pl.program_id(0) == 0)
    def _():
        amax_ref[...] = jnp.full((1, 128), bm, jnp.float32)

    @pl.when(pl.program_id(0) != 0)
    def _():
        amax_ref[...] = jnp.maximum(amax_ref[...], bm)


def _tc2(u, s1, s2, bn1_w, bn1_b, style, ad2_w1, ad2_b1, ad2_w2, ad2_b2,
         gat_w, att_src, att_dst):
    return pl.pallas_call(
        _tc2_body,
        grid=(N_OUT // BLK,),
        in_specs=[
            pl.BlockSpec((BLK, H), lambda i: (i, 0)),
            pl.BlockSpec((1, H), lambda i: (0, 0)),
            pl.BlockSpec((1, H), lambda i: (0, 0)),
            pl.BlockSpec((H,), lambda i: (0,)),
            pl.BlockSpec((H,), lambda i: (0,)),
            pl.BlockSpec((BLK, D), lambda i: (i, 0)),
            pl.BlockSpec((H, D), lambda i: (0, 0)),
            pl.BlockSpec((H,), lambda i: (0,)),
            pl.BlockSpec((H, D), lambda i: (0, 0)),
            pl.BlockSpec((H,), lambda i: (0,)),
            pl.BlockSpec((D, H), lambda i: (0, 0)),
            pl.BlockSpec((D,), lambda i: (0,)),
            pl.BlockSpec((D,), lambda i: (0,)),
        ],
        out_specs=[
            pl.BlockSpec((BLK, DP), lambda i: (i, 0)),
            pl.BlockSpec((BLK,), lambda i: (i,)),
            pl.BlockSpec((BLK,), lambda i: (i,)),
            pl.BlockSpec((1, 128), lambda i: (0, 0)),
        ],
        out_shape=[
            jax.ShapeDtypeStruct((N_OUT, DP), jnp.float32),
            jax.ShapeDtypeStruct((N_OUT,), jnp.float32),
            jax.ShapeDtypeStruct((N_OUT,), jnp.float32),
            jax.ShapeDtypeStruct((1, 128), jnp.float32),
        ],
    )(u, s1, s2, bn1_w, bn1_b, style, ad2_w1, ad2_b1, ad2_w2, ad2_b2,
      gat_w, att_src, att_dst)


def _sc_edge_body(src_hbm, dst_hbm, adst_hbm, amax_hbm, hp_hbm,
                  agg_out,
                  src_w, dst_w, dsts_w, adst_t, amax_t, rows_a, rows_b, rows_c,
                  agg_sh, sg0, sg1, sg2, ss0, ss1, ss2, si0, si1, si2):
    cid = lax.axis_index("c")
    sid = lax.axis_index("s")
    wid = cid * NS + sid
    rowbase = wid * NWIN
    z16 = jnp.zeros((16,), jnp.float32)

    pltpu.sync_copy(adst_hbm, adst_t)
    pltpu.sync_copy(amax_hbm.at[0, pl.ds(0, 16)], amax_t)

    def _zrow(j, carry):
        for q in range(DP // 16):
            rows_a[j, pl.ds(q * 16, 16)] = z16
        return carry
    lax.fori_loop(0, W, _zrow, 0)

    for k in range(RSUB // W):
        pltpu.sync_copy(rows_a, agg_sh.at[pl.ds(sid * RSUB + k * W, W)])
    plsc.subcore_barrier()
    av = amax_t[...]

    rows = (rows_a, rows_b, rows_c)
    sg = (sg0, sg1, sg2)
    ss = (ss0, ss1, ss2)
    si = (si0, si1, si2)

    def _idx_issue(slot, row, sem):
        pltpu.async_copy(src_hbm.at[pl.ds(row, 1)],
                         src_w.at[pl.ds(slot, 1)], sem)
        pltpu.async_copy(dst_hbm.at[pl.ds(row, 1)],
                         dst_w.at[pl.ds(slot, 1)], sem)

    def _idx_wait(slot, row, sem):
        pltpu.make_async_copy(src_hbm.at[pl.ds(row, 1)],
                              src_w.at[pl.ds(slot, 1)], sem).wait()
        pltpu.make_async_copy(dst_hbm.at[pl.ds(row, 1)],
                              dst_w.at[pl.ds(slot, 1)], sem).wait()

    def _gath_issue(slot):
        pltpu.async_copy(hp_hbm.at[src_w.at[slot]], rows[slot], sg[slot])

    def _gath_wait(slot):
        pltpu.make_async_copy(hp_hbm.at[src_w.at[slot]], rows[slot],
                              sg[slot]).wait()

    def _scat_issue(slot):
        pltpu.async_copy(rows[slot], agg_sh.at[dsts_w.at[slot]], ss[slot],
                         add=True)

    def _scat_wait(slot):
        pltpu.make_async_copy(rows[slot], agg_sh.at[dsts_w.at[slot]],
                              ss[slot]).wait()

    def _compute(slot):
        rv = rows[slot]
        c65 = jnp.full((16,), D + 1, jnp.int32)

        def _vreg(k2, carry):
            o = pl.multiple_of(k2 * 16, 16)
            idx_d = dst_w[slot, pl.ds(o, 16)]
            dsts_w[slot, pl.ds(o, 16)] = idx_d
            ridx = lax.iota(jnp.int32, 16) + o
            s = plsc.load_gather(rv, [ridx, c65])
            t = plsc.load_gather(adst_t, [idx_d])
            stv = s + t
            e = jnp.where(stv >= 0, stv, 0.2 * stv)
            bv = av + t
            b = jnp.where(bv >= 0, bv, 0.2 * bv)
            ex = jnp.exp(e - b)
            for l in range(16):
                wv = jnp.full((16,), ex[l], jnp.float32)
                for q in range(5):
                    rv[o + l, pl.ds(q * 16, 16)] = (
                        rv[o + l, pl.ds(q * 16, 16)] * wv)
            return carry
        lax.fori_loop(0, W // 16, _vreg, 0)

    def _tri(t, first, last):
        r0 = rowbase + 3 * t
        _idx_wait(1, r0 + 1, si[1])
        if not first:
            _scat_wait(1)
        _gath_issue(1)
        _gath_wait(0)
        _compute(0)
        _scat_issue(0)
        if not last:
            _idx_issue(0, r0 + 3, si[0])
        _idx_wait(2, r0 + 2, si[2])
        if not first:
            _scat_wait(2)
        _gath_issue(2)
        _gath_wait(1)
        _compute(1)
        _scat_issue(1)
        if not last:
            _idx_issue(1, r0 + 4, si[1])
        _scat_wait(0)
        if not last:
            _idx_wait(0, r0 + 3, si[0])
            _gath_issue(0)
        _gath_wait(2)
        _compute(2)
        _scat_issue(2)
        if not last:
            _idx_issue(2, r0 + 5, si[2])

    pltpu.sync_copy(src_hbm.at[pl.ds(rowbase, 1)], src_w.at[pl.ds(0, 1)])
    pltpu.sync_copy(dst_hbm.at[pl.ds(rowbase, 1)], dst_w.at[pl.ds(0, 1)])
    _gath_issue(0)
    _idx_issue(1, rowbase + 1, si[1])
    _idx_issue(2, rowbase + 2, si[2])

    _tri(0, True, False)

    def _body(t, carry):
        _tri(t, False, False)
        return carry
    lax.fori_loop(1, NTRI - 1, _body, 0)

    _tri(NTRI - 1, False, True)

    tb = rowbase + 3 * NTRI
    pltpu.sync_copy(src_hbm.at[pl.ds(tb, 1)], src_w.at[pl.ds(0, 1)])
    pltpu.sync_copy(dst_hbm.at[pl.ds(tb, 1)], dst_w.at[pl.ds(0, 1)])
    _gath_issue(0)
    pltpu.sync_copy(src_hbm.at[pl.ds(tb + 1, 1)], src_w.at[pl.ds(1, 1)])
    pltpu.sync_copy(dst_hbm.at[pl.ds(tb + 1, 1)], dst_w.at[pl.ds(1, 1)])
    _scat_wait(1)
    _gath_issue(1)
    _gath_wait(0)
    _compute(0)
    _scat_issue(0)
    _gath_wait(1)
    _compute(1)
    _scat_issue(1)
    _scat_wait(0)
    _scat_wait(1)
    _scat_wait(2)

    plsc.subcore_barrier()

    out_base = cid * N_OUT + sid * RSUB
    pltpu.sync_copy(agg_sh.at[pl.ds(sid * RSUB, RSUB)],
                    agg_out.at[pl.ds(out_base, RSUB)])


def _sc_edge(src2d, dst2d, a_dst, amax, hp_pad):
    mesh = plsc.VectorSubcoreMesh(core_axis_name="c", subcore_axis_name="s",
                                  num_cores=NC, num_subcores=NS)
    return pl.kernel(
        _sc_edge_body,
        out_type=jax.ShapeDtypeStruct((NC * N_OUT, DP), jnp.float32),
        mesh=mesh,
        compiler_params=pltpu.CompilerParams(needs_layout_passes=False),
        scratch_types=[
            pltpu.VMEM((3, 128), jnp.int32),
            pltpu.VMEM((3, 128), jnp.int32),
            pltpu.VMEM((3, 128), jnp.int32),
            pltpu.VMEM((N_OUT,), jnp.float32),
            pltpu.VMEM((16,), jnp.float32),
            pltpu.VMEM((W, DP), jnp.float32),
            pltpu.VMEM((W, DP), jnp.float32),
            pltpu.VMEM((W, DP), jnp.float32),
            pltpu.VMEM_SHARED((N_OUT, DP), jnp.float32),
            pltpu.SemaphoreType.DMA,
            pltpu.SemaphoreType.DMA,
            pltpu.SemaphoreType.DMA,
            pltpu.SemaphoreType.DMA,
            pltpu.SemaphoreType.DMA,
            pltpu.SemaphoreType.DMA,
            pltpu.SemaphoreType.DMA,
            pltpu.SemaphoreType.DMA,
            pltpu.SemaphoreType.DMA,
        ],
    )(src2d, dst2d, a_dst, amax, hp_pad)


def _tc3_body(agg0_ref, agg1_ref, hp_ref, asrc_ref, adst_ref, amax_ref,
              gatb_ref, style_ref, a3w1_ref, a3b1_ref, a3w2_ref, a3b2_ref,
              out_ref):
    es = _leaky(asrc_ref[...] + adst_ref[...])
    btab = _leaky(amax_ref[0, 0] + adst_ref[...])
    ex_self = jnp.exp(es - btab)
    hp = hp_ref[...][:, :D]
    den = agg0_ref[...][:, D] + agg1_ref[...][:, D] + ex_self
    aggr = agg0_ref[...][:, :D] + agg1_ref[...][:, :D] + ex_self[:, None] * hp
    agg = aggr / (den[:, None] + 1e-16) + gatb_ref[...]
    h2 = _leaky(agg)
    rm = jnp.mean(h2, axis=1, keepdims=True)
    rs = jnp.sqrt(jnp.sum((h2 - rm) ** 2, axis=1, keepdims=True) / (D - 1))
    st = style_ref[...]
    gamma = jnp.dot(st, a3w1_ref[...].T, preferred_element_type=jnp.float32) + a3b1_ref[...]
    beta = jnp.dot(st, a3w2_ref[...].T, preferred_element_type=jnp.float32) + a3b2_ref[...]
    out_ref[...] = gamma * (h2 - rm) / (rs + 1e-8) + beta


def _tc3(agg, hp_pad, a_src, a_dst, amax, gat_b, style,
         ad3_w1, ad3_b1, ad3_w2, ad3_b2):
    nb = N_OUT // BLK
    return pl.pallas_call(
        _tc3_body,
        grid=(nb,),
        in_specs=[
            pl.BlockSpec((BLK, DP), lambda i: (i, 0)),
            pl.BlockSpec((BLK, DP), lambda i: (i + N_OUT // BLK, 0)),
            pl.BlockSpec((BLK, DP), lambda i: (i, 0)),
            pl.BlockSpec((BLK,), lambda i: (i,)),
            pl.BlockSpec((BLK,), lambda i: (i,)),
            pl.BlockSpec((1, 128), lambda i: (0, 0)),
            pl.BlockSpec((D,), lambda i: (0,)),
            pl.BlockSpec((BLK, D), lambda i: (i, 0)),
            pl.BlockSpec((D, D), lambda i: (0, 0)),
            pl.BlockSpec((D,), lambda i: (0,)),
            pl.BlockSpec((D, D), lambda i: (0, 0)),
            pl.BlockSpec((D,), lambda i: (0,)),
        ],
        out_specs=pl.BlockSpec((BLK, D), lambda i: (i, 0)),
        out_shape=jax.ShapeDtypeStruct((N_OUT, D), jnp.float32),
    )(agg, agg, hp_pad, a_src, a_dst, amax, gat_b, style,
      ad3_w1, ad3_b1, ad3_w2, ad3_b2)


def kernel(x, edge_index, style, trs_w, trs_b, bn2_w, bn2_b, ad1_w1, ad1_b1,
           ad1_w2, ad1_b2, fc1_w, fc1_b, bn1_w, bn1_b, ad2_w1, ad2_b1,
           ad2_w2, ad2_b2, gat_w, gat_att_src, gat_att_dst, gat_b, ad3_w1,
           ad3_b1, ad3_w2, ad3_b2):
    u, s1, s2 = _tc1(x, trs_w, trs_b, bn2_w, bn2_b, style, ad1_w1, ad1_b1,
                     ad1_w2, ad1_b2, fc1_w, fc1_b)
    hp_pad, a_src, a_dst, amax = _tc2(u, s1, s2, bn1_w, bn1_b, style, ad2_w1,
                                      ad2_b1, ad2_w2, ad2_b2, gat_w,
                                      gat_att_src, gat_att_dst)
    src2d = edge_index[0].reshape(E // 128, 128)
    dst2d = edge_index[1].reshape(E // 128, 128)
    agg = _sc_edge(src2d, dst2d, a_dst, amax, hp_pad)
    return _tc3(agg, hp_pad, a_src, a_dst, amax, gat_b, style,
                ad3_w1, ad3_b1, ad3_w2, ad3_b2)
